# Initial kernel scaffold; baseline (speedup 1.0000x reference)
import jax
import jax.numpy as jnp
from jax.experimental import pallas as pl


def kernel(input_ids, word_embeddings, position_embeddings, ln_weight, ln_bias):
    raise NotImplementedError("write your pallas kernel here")


# speedup vs baseline: 2.7768x; 2.7768x over previous
import functools

import jax
import jax.numpy as jnp
from jax import lax
from jax.experimental import pallas as pl
from jax.experimental.pallas import tpu as pltpu
from jax.experimental.pallas import tpu_sc as plsc

VOCAB = 50265
HIDDEN = 768
BATCH = 64
SEQ = 512
EPS = 1e-12

NUM_WORKERS = 32
TOKENS = BATCH * SEQ
TOK_PER_W = TOKENS // NUM_WORKERS
CHUNK = 64
NCHUNK = TOK_PER_W // CHUNK


def _sc_gather(table, ids):
    mesh = plsc.VectorSubcoreMesh(core_axis_name="c", subcore_axis_name="s")

    @functools.partial(
        pl.kernel,
        out_type=jax.ShapeDtypeStruct((TOKENS, HIDDEN), jnp.float32),
        mesh=mesh,
        scratch_types=[
            pltpu.VMEM((TOK_PER_W,), jnp.int32),
            pltpu.VMEM((CHUNK, HIDDEN), jnp.float32),
            pltpu.VMEM((CHUNK, HIDDEN), jnp.float32),
            pltpu.SemaphoreType.DMA,
            pltpu.SemaphoreType.DMA,
        ],
    )
    def gather_kernel(table_hbm, idx_hbm, out_hbm, idx_v, rows0, rows1, g0, g1):
        wid = lax.axis_index("s") * 2 + lax.axis_index("c")
        base = wid * TOK_PER_W
        pltpu.sync_copy(idx_hbm.at[pl.ds(base, TOK_PER_W)], idx_v)

        bufs = (rows0, rows1)
        sems = (g0, g1)

        for b in range(2):
            pltpu.async_copy(
                table_hbm.at[idx_v.at[pl.ds(b * CHUNK, CHUNK)]], bufs[b], sems[b]
            )

        @pl.loop(0, NCHUNK, step=2)
        def _(i):
            for b in range(2):
                cur = i + b
                pltpu.make_async_copy(
                    table_hbm.at[idx_v.at[pl.ds(cur * CHUNK, CHUNK)]],
                    bufs[b],
                    sems[b],
                ).wait()
                pltpu.sync_copy(bufs[b], out_hbm.at[pl.ds(base + cur * CHUNK, CHUNK)])
                nxt = cur + 2

                @pl.when(nxt < NCHUNK)
                def _():
                    pltpu.async_copy(
                        table_hbm.at[idx_v.at[pl.ds(nxt * CHUNK, CHUNK)]],
                        bufs[b],
                        sems[b],
                    )

    return gather_kernel(table, ids)


def _ln_body(x_ref, pos_ref, w_ref, b_ref, o_ref):
    x = x_ref[0] + pos_ref[...]
    u = jnp.mean(x, axis=1, keepdims=True)
    d = x - u
    s = jnp.mean(d * d, axis=1, keepdims=True)
    y = d * lax.rsqrt(s + EPS)
    y = y * w_ref[...] + b_ref[...]
    o_ref[0] = y.T


def _ln_transpose(gathered, pos, w, b):
    grid = (BATCH,)
    return pl.pallas_call(
        _ln_body,
        grid=grid,
        in_specs=[
            pl.BlockSpec((1, SEQ, HIDDEN), lambda i: (i, 0, 0)),
            pl.BlockSpec((SEQ, HIDDEN), lambda i: (0, 0)),
            pl.BlockSpec((1, HIDDEN), lambda i: (0, 0)),
            pl.BlockSpec((1, HIDDEN), lambda i: (0, 0)),
        ],
        out_specs=pl.BlockSpec((1, HIDDEN, SEQ), lambda i: (i, 0, 0)),
        out_shape=jax.ShapeDtypeStruct((BATCH, HIDDEN, SEQ), jnp.float32),
    )(gathered, pos, w, b)


@jax.jit
def kernel(input_ids, word_embeddings, position_embeddings, ln_weight, ln_bias):
    ids = input_ids.reshape(-1).astype(jnp.int32)
    gathered = _sc_gather(word_embeddings, ids)
    gathered = gathered.reshape(BATCH, SEQ, HIDDEN)
    pos = position_embeddings[:SEQ]
    w = ln_weight.reshape(1, HIDDEN)
    b = ln_bias.reshape(1, HIDDEN)
    return _ln_transpose(gathered, pos, w, b)


# speedup vs baseline: 2.7811x; 1.0015x over previous
import functools

import jax
import jax.numpy as jnp
from jax import lax
from jax.experimental import pallas as pl
from jax.experimental.pallas import tpu as pltpu
from jax.experimental.pallas import tpu_sc as plsc

VOCAB = 50265
HIDDEN = 768
BATCH = 64
SEQ = 512
EPS = 1e-12

NUM_WORKERS = 32
TOKENS = BATCH * SEQ
TOK_PER_W = TOKENS // NUM_WORKERS
CHUNK = 64
NCHUNK = TOK_PER_W // CHUNK


def _sc_gather(table, ids):
    mesh = plsc.VectorSubcoreMesh(core_axis_name="c", subcore_axis_name="s")

    @functools.partial(
        pl.kernel,
        out_type=jax.ShapeDtypeStruct((TOKENS, HIDDEN), jnp.float32),
        mesh=mesh,
        scratch_types=[
            pltpu.VMEM((TOK_PER_W,), jnp.int32),
            pltpu.VMEM((CHUNK, HIDDEN), jnp.float32),
            pltpu.VMEM((CHUNK, HIDDEN), jnp.float32),
            pltpu.SemaphoreType.DMA,
            pltpu.SemaphoreType.DMA,
        ],
    )
    def gather_kernel(table_hbm, idx_hbm, out_hbm, idx_v, rows0, rows1, g0, g1):
        wid = lax.axis_index("s") * 2 + lax.axis_index("c")
        base = wid * TOK_PER_W
        pltpu.sync_copy(idx_hbm.at[pl.ds(base, TOK_PER_W)], idx_v)

        bufs = (rows0, rows1)
        sems = (g0, g1)

        for b in range(2):
            pltpu.async_copy(
                table_hbm.at[idx_v.at[pl.ds(b * CHUNK, CHUNK)]], bufs[b], sems[b]
            )

        @pl.loop(0, NCHUNK, step=2)
        def _(i):
            for b in range(2):
                cur = i + b
                pltpu.make_async_copy(
                    table_hbm.at[idx_v.at[pl.ds(cur * CHUNK, CHUNK)]],
                    bufs[b],
                    sems[b],
                ).wait()
                pltpu.sync_copy(bufs[b], out_hbm.at[pl.ds(base + cur * CHUNK, CHUNK)])
                nxt = cur + 2

                @pl.when(nxt < NCHUNK)
                def _():
                    pltpu.async_copy(
                        table_hbm.at[idx_v.at[pl.ds(nxt * CHUNK, CHUNK)]],
                        bufs[b],
                        sems[b],
                    )

    return gather_kernel(table, ids)


def _ln_body(x_ref, pos_ref, w_ref, b_ref, o_ref):
    x = x_ref[0] + pos_ref[...]
    u = jnp.mean(x, axis=1, keepdims=True)
    d = x - u
    s = jnp.mean(d * d, axis=1, keepdims=True)
    y = d * lax.rsqrt(s + EPS)
    y = y * w_ref[...] + b_ref[...]
    o_ref[0] = y.T


def _ln_transpose(gathered, pos, w, b):
    grid = (BATCH,)
    return pl.pallas_call(
        _ln_body,
        grid=grid,
        in_specs=[
            pl.BlockSpec((1, SEQ, HIDDEN), lambda i: (i, 0, 0)),
            pl.BlockSpec((SEQ, HIDDEN), lambda i: (0, 0)),
            pl.BlockSpec((1, HIDDEN), lambda i: (0, 0)),
            pl.BlockSpec((1, HIDDEN), lambda i: (0, 0)),
        ],
        out_specs=pl.BlockSpec((1, HIDDEN, SEQ), lambda i: (i, 0, 0)),
        out_shape=jax.ShapeDtypeStruct((BATCH, HIDDEN, SEQ), jnp.float32),
        compiler_params=pltpu.CompilerParams(
            dimension_semantics=("parallel",),
        ),
    )(gathered, pos, w, b)


@jax.jit
def kernel(input_ids, word_embeddings, position_embeddings, ln_weight, ln_bias):
    ids = input_ids.reshape(-1).astype(jnp.int32)
    gathered = _sc_gather(word_embeddings, ids)
    gathered = gathered.reshape(BATCH, SEQ, HIDDEN)
    pos = position_embeddings[:SEQ]
    w = ln_weight.reshape(1, HIDDEN)
    b = ln_bias.reshape(1, HIDDEN)
    return _ln_transpose(gathered, pos, w, b)


# speedup vs baseline: 3.2600x; 1.1722x over previous
import functools

import jax
import jax.numpy as jnp
from jax import lax
from jax.experimental import pallas as pl
from jax.experimental.pallas import tpu as pltpu
from jax.experimental.pallas import tpu_sc as plsc

VOCAB = 50265
HIDDEN = 768
BATCH = 64
SEQ = 512
EPS = 1e-12

NUM_WORKERS = 32
TOKENS = BATCH * SEQ
TOK_PER_W = TOKENS // NUM_WORKERS
CHUNK = 64
NCHUNK = TOK_PER_W // CHUNK


def _sc_gather(table, ids):
    mesh = plsc.VectorSubcoreMesh(core_axis_name="c", subcore_axis_name="s")

    @functools.partial(
        pl.kernel,
        out_type=jax.ShapeDtypeStruct((TOKENS, HIDDEN), jnp.float32),
        mesh=mesh,
        scratch_types=[
            pltpu.VMEM((TOK_PER_W,), jnp.int32),
            pltpu.VMEM((CHUNK, HIDDEN), jnp.float32),
            pltpu.VMEM((CHUNK, HIDDEN), jnp.float32),
            pltpu.SemaphoreType.DMA,
            pltpu.SemaphoreType.DMA,
        ],
    )
    def gather_kernel(table_hbm, idx_hbm, out_hbm, idx_v, rows0, rows1, g0, g1):
        wid = lax.axis_index("s") * 2 + lax.axis_index("c")
        base = wid * TOK_PER_W
        pltpu.sync_copy(idx_hbm.at[pl.ds(base, TOK_PER_W)], idx_v)

        bufs = (rows0, rows1)
        sems = (g0, g1)

        for b in range(2):
            pltpu.async_copy(
                table_hbm.at[idx_v.at[pl.ds(b * CHUNK, CHUNK)]], bufs[b], sems[b]
            )

        @pl.loop(0, NCHUNK, step=2)
        def _(i):
            for b in range(2):
                cur = i + b
                pltpu.make_async_copy(
                    table_hbm.at[idx_v.at[pl.ds(cur * CHUNK, CHUNK)]],
                    bufs[b],
                    sems[b],
                ).wait()
                pltpu.sync_copy(bufs[b], out_hbm.at[pl.ds(base + cur * CHUNK, CHUNK)])
                nxt = cur + 2

                @pl.when(nxt < NCHUNK)
                def _():
                    pltpu.async_copy(
                        table_hbm.at[idx_v.at[pl.ds(nxt * CHUNK, CHUNK)]],
                        bufs[b],
                        sems[b],
                    )

    return gather_kernel(table, ids)


ROWS_BLK = 4


def _ln_body(x_ref, pos_ref, w_ref, b_ref, o_ref):
    for r in range(ROWS_BLK):
        x = x_ref[r] + pos_ref[...]
        u = jnp.mean(x, axis=1, keepdims=True)
        d = x - u
        s = jnp.mean(d * d, axis=1, keepdims=True)
        y = d * lax.rsqrt(s + EPS)
        y = y * w_ref[...] + b_ref[...]
        o_ref[r] = y.T


def _ln_transpose(gathered, pos, w, b):
    grid = (BATCH // ROWS_BLK,)
    return pl.pallas_call(
        _ln_body,
        grid=grid,
        in_specs=[
            pl.BlockSpec((ROWS_BLK, SEQ, HIDDEN), lambda i: (i, 0, 0)),
            pl.BlockSpec((SEQ, HIDDEN), lambda i: (0, 0)),
            pl.BlockSpec((1, HIDDEN), lambda i: (0, 0)),
            pl.BlockSpec((1, HIDDEN), lambda i: (0, 0)),
        ],
        out_specs=pl.BlockSpec((ROWS_BLK, HIDDEN, SEQ), lambda i: (i, 0, 0)),
        out_shape=jax.ShapeDtypeStruct((BATCH, HIDDEN, SEQ), jnp.float32),
        compiler_params=pltpu.CompilerParams(
            dimension_semantics=("parallel",),
        ),
    )(gathered, pos, w, b)


@jax.jit
def kernel(input_ids, word_embeddings, position_embeddings, ln_weight, ln_bias):
    ids = input_ids.reshape(-1).astype(jnp.int32)
    gathered = _sc_gather(word_embeddings, ids)
    gathered = gathered.reshape(BATCH, SEQ, HIDDEN)
    pos = position_embeddings[:SEQ]
    w = ln_weight.reshape(1, HIDDEN)
    b = ln_bias.reshape(1, HIDDEN)
    return _ln_transpose(gathered, pos, w, b)
